# baseline (speedup 1.0000x reference)
import functools

import jax
import jax.numpy as jnp
from jax import lax
from jax.experimental import pallas as pl
from jax.experimental.pallas import tpu as pltpu
from jax.experimental.pallas import tpu_sc as plsc

NC = 2
NS = 16
NW = NC * NS
L = 16

N_PAD = 10240
CHUNK = 64
RPT = N_PAD // NS
NBUF = 3

CHUNK1, NBUF1 = 80, 3
CHUNK2, NBUF2 = 128, 3
SPLIT1 = (61, 23)
SPLIT2 = (39, 15)

_mesh = plsc.VectorSubcoreMesh(core_axis_name="c", subcore_axis_name="s")
_sc_params = pltpu.CompilerParams(use_tc_tiling_on_sc=False)


def _make_deg_kernel(e_pad):
    ew = e_pad // NW
    n_iter = ew // CHUNK

    @functools.partial(
        pl.kernel,
        out_type=jax.ShapeDtypeStruct((NC, N_PAD), jnp.float32),
        mesh=_mesh,
        scratch_types=[
            pltpu.VMEM((n_iter, CHUNK), jnp.int32),
            pltpu.VMEM((CHUNK,), jnp.float32),
            pltpu.VMEM((RPT,), jnp.float32),
            pltpu.VMEM_SHARED((N_PAD,), jnp.float32),
            pltpu.SemaphoreType.DMA,
        ],
    )
    def deg_kernel(dst_hbm, out_hbm, idx_v, ones_v, zrow_v, acc_sh, sem):
        c = lax.axis_index("c")
        s = lax.axis_index("s")
        wid = s * NC + c

        one = jnp.full((L,), 1.0, jnp.float32)
        z = jnp.zeros((L,), jnp.float32)
        for g in range(CHUNK // L):
            ones_v[pl.ds(g * L, L)] = one
        for g in range(RPT // L):
            zrow_v[pl.ds(g * L, L)] = z
        pltpu.sync_copy(dst_hbm.at[wid], idx_v)
        pltpu.sync_copy(zrow_v, acc_sh.at[pl.ds(s * RPT, RPT)])
        plsc.subcore_barrier()

        def fire(k, _):
            pltpu.async_copy(ones_v, acc_sh.at[idx_v.at[k]], sem, add=True)
            return 0

        def drain(k, _):
            pltpu.make_async_copy(ones_v, acc_sh.at[idx_v.at[0]], sem).wait()
            return 0

        lax.fori_loop(0, n_iter, fire, 0)
        lax.fori_loop(0, n_iter, drain, 0)
        plsc.subcore_barrier()
        pltpu.sync_copy(acc_sh.at[pl.ds(s * RPT, RPT)],
                        out_hbm.at[c, pl.ds(s * RPT, RPT)])

    return deg_kernel


def _make_agg_kernel(d, chunk, nbuf, n_g0, n_g1):
    CHUNK = chunk
    NBUF = nbuf
    n_gmax = max(n_g0, n_g1)

    @functools.partial(
        pl.kernel,
        out_type=jax.ShapeDtypeStruct((NC, N_PAD, d), jnp.float32),
        mesh=_mesh,
        scratch_types=[
            [pltpu.VMEM((2, NBUF, CHUNK), jnp.int32) for _ in range(2)],
            [pltpu.VMEM((CHUNK, d), jnp.float32) for _ in range(NBUF)],
            pltpu.VMEM_SHARED((N_PAD, d), jnp.float32),
            [pltpu.SemaphoreType.DMA for _ in range(2)],
            [pltpu.SemaphoreType.DMA for _ in range(NBUF)],
            [pltpu.SemaphoreType.DMA for _ in range(NBUF)],
        ],
        compiler_params=_sc_params,
    )
    def agg_kernel(y_hbm, pk_hbm, out_hbm, ibuf, rows, acc_sh,
                   isem, gsem, ssem):
        c = lax.axis_index("c")
        s = lax.axis_index("s")

        z = jnp.zeros((L,), jnp.float32)
        for r in range(CHUNK):
            for g in range(d // L):
                rows[0][r, pl.ds(g * L, L)] = z

        for j in range(RPT // CHUNK):
            pltpu.sync_copy(rows[0], acc_sh.at[pl.ds(s * RPT + j * CHUNK, CHUNK)])
        plsc.subcore_barrier()

        def group_step(j, par):
            pltpu.async_copy(pk_hbm.at[c, s, j + 1], ibuf[1 - par],
                             isem[1 - par])
            for b in range(NBUF):
                pltpu.make_async_copy(
                    y_hbm.at[ibuf[par].at[0, b]], rows[b], gsem[b]).wait()
                pltpu.async_copy(
                    rows[b], acc_sh.at[ibuf[par].at[1, b]], ssem[b], add=True)
            pltpu.make_async_copy(pk_hbm.at[c, s, 0], ibuf[1 - par],
                                  isem[1 - par]).wait()
            for b in range(NBUF):
                pltpu.make_async_copy(
                    rows[b], acc_sh.at[ibuf[par].at[1, b]], ssem[b]).wait()
                pltpu.async_copy(
                    y_hbm.at[ibuf[1 - par].at[0, b]], rows[b], gsem[b])

        def run_groups(n_g):
            pltpu.sync_copy(pk_hbm.at[c, s, 0], ibuf[0])
            for b in range(NBUF):
                pltpu.async_copy(y_hbm.at[ibuf[0].at[0, b]], rows[b], gsem[b])

            def body(u, _):
                group_step(2 * u, 0)
                group_step(2 * u + 1, 1)
                return 0

            lax.fori_loop(0, (n_g - 1) // 2, body, 0)

            for b in range(NBUF):
                pltpu.make_async_copy(
                    y_hbm.at[ibuf[0].at[0, b]], rows[b], gsem[b]).wait()
                pltpu.async_copy(
                    rows[b], acc_sh.at[ibuf[0].at[1, b]], ssem[b], add=True)
            for b in range(NBUF):
                pltpu.make_async_copy(
                    rows[b], acc_sh.at[ibuf[0].at[1, b]], ssem[b]).wait()

        if n_g0 == n_g1:
            run_groups(n_g0)
        else:
            @pl.when(c == 0)
            def _():
                run_groups(n_g0)

            @pl.when(c == 1)
            def _():
                run_groups(n_g1)

        plsc.subcore_barrier()
        pltpu.sync_copy(acc_sh.at[pl.ds(s * RPT, RPT)],
                        out_hbm.at[c, pl.ds(s * RPT, RPT)])

    return agg_kernel


_BLK = 1024


def _mm1a_body(x_ref, w_ref, xw_ref):
    xw_ref[...] = jnp.dot(x_ref[...], w_ref[...],
                          preferred_element_type=jnp.float32)


def _mm1b_body(xw_ref, deg_ref, y_ref, dinv_ref):
    dg = deg_ref[...]
    dinv = lax.rsqrt(dg[0] + dg[1] + 1.0)
    y_ref[...] = dinv * xw_ref[...]
    dinv_ref[...] = dinv


def _mm2_body(p_ref, y1_ref, dinv_ref, b1_ref, w2_ref, y2_ref):
    pr = p_ref[...]
    dinv = dinv_ref[...]
    agg = pr[0] + pr[1] + y1_ref[...]
    h = jnp.maximum(dinv * agg + b1_ref[...], 0.0)
    y2_ref[...] = dinv * jnp.dot(h, w2_ref[...], preferred_element_type=jnp.float32)


def _fin_body(q_ref, y2_ref, dinv_ref, b2_ref, o_ref):
    qr = q_ref[...]
    o_ref[...] = dinv_ref[...] * (qr[0] + qr[1] + y2_ref[...]) + b2_ref[...]


def _mm1a(xp, w1):
    g = N_PAD // _BLK
    din = xp.shape[1]
    return pl.pallas_call(
        _mm1a_body,
        grid=(g,),
        in_specs=[
            pl.BlockSpec((_BLK, din), lambda i: (i, 0)),
            pl.BlockSpec((din, w1.shape[1]), lambda i: (0, 0)),
        ],
        out_specs=pl.BlockSpec((_BLK, w1.shape[1]), lambda i: (i, 0)),
        out_shape=jax.ShapeDtypeStruct((N_PAD, w1.shape[1]), jnp.float32),
    )(xp, w1)


def _mm1b(xw, degp):
    g = N_PAD // _BLK
    hid = xw.shape[1]
    return pl.pallas_call(
        _mm1b_body,
        grid=(g,),
        in_specs=[
            pl.BlockSpec((_BLK, hid), lambda i: (i, 0)),
            pl.BlockSpec((NC, _BLK, 1), lambda i: (0, i, 0)),
        ],
        out_specs=[
            pl.BlockSpec((_BLK, hid), lambda i: (i, 0)),
            pl.BlockSpec((_BLK, 1), lambda i: (i, 0)),
        ],
        out_shape=[
            jax.ShapeDtypeStruct((N_PAD, hid), jnp.float32),
            jax.ShapeDtypeStruct((N_PAD, 1), jnp.float32),
        ],
    )(xw, degp)


def _mm2(p, y1, dinv, b1, w2p):
    g = N_PAD // _BLK
    h = y1.shape[1]
    d2 = w2p.shape[1]
    return pl.pallas_call(
        _mm2_body,
        grid=(g,),
        in_specs=[
            pl.BlockSpec((NC, _BLK, h), lambda i: (0, i, 0)),
            pl.BlockSpec((_BLK, h), lambda i: (i, 0)),
            pl.BlockSpec((_BLK, 1), lambda i: (i, 0)),
            pl.BlockSpec((1, h), lambda i: (0, 0)),
            pl.BlockSpec((h, d2), lambda i: (0, 0)),
        ],
        out_specs=pl.BlockSpec((_BLK, d2), lambda i: (i, 0)),
        out_shape=jax.ShapeDtypeStruct((N_PAD, d2), jnp.float32),
    )(p, y1, dinv, b1, w2p)


def _fin(q, y2, dinv, b2p):
    g = N_PAD // _BLK
    d2 = y2.shape[1]
    return pl.pallas_call(
        _fin_body,
        grid=(g,),
        in_specs=[
            pl.BlockSpec((NC, _BLK, d2), lambda i: (0, i, 0)),
            pl.BlockSpec((_BLK, d2), lambda i: (i, 0)),
            pl.BlockSpec((_BLK, 1), lambda i: (i, 0)),
            pl.BlockSpec((1, d2), lambda i: (0, 0)),
        ],
        out_specs=pl.BlockSpec((_BLK, d2), lambda i: (i, 0)),
        out_shape=jax.ShapeDtypeStruct((N_PAD, d2), jnp.float32),
    )(q, y2, dinv, b2p)


@jax.jit
def kernel(x, edge_index, W1, b1, W2, b2):
    n, din = x.shape
    hid = W1.shape[1]
    ncls = W2.shape[1]
    e = edge_index.shape[1]

    src = edge_index[0].astype(jnp.int32)
    dst = edge_index[1].astype(jnp.int32)

    def pack(chunk, nbuf, n_g0, n_g1):
        n_gmax = max(n_g0, n_g1)
        e0 = NS * n_g0 * nbuf * chunk
        e1 = NS * n_g1 * nbuf * chunk
        assert e0 + e1 >= e

        def part(a):
            ap = jnp.concatenate([a, jnp.full((e0 + e1 - e,), n, jnp.int32)])
            a0 = ap[:e0].reshape(NS, n_g0, nbuf, chunk)
            a1 = ap[e0:].reshape(NS, n_g1, nbuf, chunk)
            a0 = jnp.pad(a0, ((0, 0), (0, n_gmax - n_g0), (0, 0), (0, 0)))
            a1 = jnp.pad(a1, ((0, 0), (0, n_gmax - n_g1), (0, 0), (0, 0)))
            return jnp.stack([a0, a1])

        return jnp.stack([part(src), part(dst)], axis=3)

    pk1 = pack(CHUNK1, NBUF1, *SPLIT1)
    pk2 = pack(CHUNK2, NBUF2, *SPLIT2)
    dquant = NW * CHUNK
    e_padd = ((e + dquant - 1) // dquant) * dquant
    dpad = jnp.concatenate([dst, jnp.full((e_padd - e,), n, jnp.int32)])
    dstd = dpad.reshape(NW, e_padd // (NW * CHUNK), CHUNK)

    xp = jnp.zeros((N_PAD, din), jnp.float32).at[:n].set(x)
    d2 = 64
    w2p = jnp.zeros((hid, d2), jnp.float32).at[:, :ncls].set(W2)
    b1r = b1.reshape(1, hid)
    b2p = jnp.zeros((1, d2), jnp.float32).at[0, :ncls].set(b2)

    xw = _mm1a(xp, W1)
    degp = _make_deg_kernel(e_padd)(dstd)
    y1, dinv = _mm1b(xw, degp.reshape(NC, N_PAD, 1))
    p = _make_agg_kernel(hid, CHUNK1, NBUF1, *SPLIT1)(y1, pk1)
    y2 = _mm2(p, y1, dinv, b1r, w2p)
    q = _make_agg_kernel(d2, CHUNK2, NBUF2, *SPLIT2)(y2, pk2)
    o = _fin(q, y2, dinv, b2p)
    return o[:n, :ncls]


# speedup vs baseline: 1.0531x; 1.0531x over previous
import functools

import jax
import jax.numpy as jnp
from jax import lax
from jax.experimental import pallas as pl
from jax.experimental.pallas import tpu as pltpu
from jax.experimental.pallas import tpu_sc as plsc

NC = 2
NS = 16
NW = NC * NS
L = 16

N_PAD = 10240
CHUNK = 64
RPT = N_PAD // NS
NBUF = 3

CHUNK1, NBUF1 = 80, 3
CHUNK2, NBUF2 = 128, 3
SPLIT1 = (61, 23)
SPLIT2 = (39, 15)

_mesh = plsc.VectorSubcoreMesh(core_axis_name="c", subcore_axis_name="s")
_sc_params = pltpu.CompilerParams(use_tc_tiling_on_sc=False)


def _make_deg_kernel(e_pad):
    ew = e_pad // NW
    n_iter = ew // CHUNK

    @functools.partial(
        pl.kernel,
        out_type=jax.ShapeDtypeStruct((NC, N_PAD), jnp.float32),
        mesh=_mesh,
        scratch_types=[
            pltpu.VMEM((n_iter, CHUNK), jnp.int32),
            pltpu.VMEM((CHUNK,), jnp.float32),
            pltpu.VMEM((RPT,), jnp.float32),
            pltpu.VMEM_SHARED((N_PAD,), jnp.float32),
            pltpu.SemaphoreType.DMA,
        ],
    )
    def deg_kernel(dst_hbm, out_hbm, idx_v, ones_v, zrow_v, acc_sh, sem):
        c = lax.axis_index("c")
        s = lax.axis_index("s")
        wid = s * NC + c

        one = jnp.full((L,), 1.0, jnp.float32)
        z = jnp.zeros((L,), jnp.float32)
        for g in range(CHUNK // L):
            ones_v[pl.ds(g * L, L)] = one
        for g in range(RPT // L):
            zrow_v[pl.ds(g * L, L)] = z
        pltpu.sync_copy(dst_hbm.at[wid], idx_v)
        pltpu.sync_copy(zrow_v, acc_sh.at[pl.ds(s * RPT, RPT)])
        plsc.subcore_barrier()

        def fire(k, _):
            pltpu.async_copy(ones_v, acc_sh.at[idx_v.at[k]], sem, add=True)
            return 0

        def drain(k, _):
            pltpu.make_async_copy(ones_v, acc_sh.at[idx_v.at[0]], sem).wait()
            return 0

        lax.fori_loop(0, n_iter, fire, 0)
        lax.fori_loop(0, n_iter, drain, 0)
        plsc.subcore_barrier()
        pltpu.sync_copy(acc_sh.at[pl.ds(s * RPT, RPT)],
                        out_hbm.at[c, pl.ds(s * RPT, RPT)])

    return deg_kernel


def _make_agg_kernel(d, chunk, nbuf, n_g0, n_g1):
    CHUNK = chunk
    NBUF = nbuf
    n_gmax = max(n_g0, n_g1)

    @functools.partial(
        pl.kernel,
        out_type=jax.ShapeDtypeStruct((NC, N_PAD, d), jnp.float32),
        mesh=_mesh,
        scratch_types=[
            [pltpu.VMEM((2, NBUF, CHUNK), jnp.int32) for _ in range(2)],
            [pltpu.VMEM((CHUNK, d), jnp.float32) for _ in range(NBUF)],
            pltpu.VMEM_SHARED((N_PAD, d), jnp.float32),
            [pltpu.SemaphoreType.DMA for _ in range(2)],
            [pltpu.SemaphoreType.DMA for _ in range(NBUF)],
            [pltpu.SemaphoreType.DMA for _ in range(NBUF)],
        ],
        compiler_params=_sc_params,
    )
    def agg_kernel(y_hbm, pk_hbm, out_hbm, ibuf, rows, acc_sh,
                   isem, gsem, ssem):
        c = lax.axis_index("c")
        s = lax.axis_index("s")

        z = jnp.zeros((L,), jnp.float32)
        for r in range(CHUNK):
            for g in range(d // L):
                rows[0][r, pl.ds(g * L, L)] = z

        for j in range(RPT // CHUNK):
            pltpu.sync_copy(rows[0], acc_sh.at[pl.ds(s * RPT + j * CHUNK, CHUNK)])
        plsc.subcore_barrier()

        def group_step(j, par):
            pltpu.async_copy(pk_hbm.at[c, s, j + 1], ibuf[1 - par],
                             isem[1 - par])
            for b in range(NBUF):
                pltpu.make_async_copy(
                    y_hbm.at[ibuf[par].at[0, b]], rows[b], gsem[b]).wait()
                pltpu.async_copy(
                    rows[b], acc_sh.at[ibuf[par].at[1, b]], ssem[b], add=True)
            pltpu.make_async_copy(pk_hbm.at[c, s, 0], ibuf[1 - par],
                                  isem[1 - par]).wait()
            for b in range(NBUF):
                pltpu.make_async_copy(
                    rows[b], acc_sh.at[ibuf[par].at[1, b]], ssem[b]).wait()
                pltpu.async_copy(
                    y_hbm.at[ibuf[1 - par].at[0, b]], rows[b], gsem[b])

        def run_groups(n_g):
            pltpu.sync_copy(pk_hbm.at[c, s, 0], ibuf[0])
            for b in range(NBUF):
                pltpu.async_copy(y_hbm.at[ibuf[0].at[0, b]], rows[b], gsem[b])

            def body(u, _):
                group_step(2 * u, 0)
                group_step(2 * u + 1, 1)
                return 0

            lax.fori_loop(0, (n_g - 1) // 2, body, 0)

            for b in range(NBUF):
                pltpu.make_async_copy(
                    y_hbm.at[ibuf[0].at[0, b]], rows[b], gsem[b]).wait()
                pltpu.async_copy(
                    rows[b], acc_sh.at[ibuf[0].at[1, b]], ssem[b], add=True)
            for b in range(NBUF):
                pltpu.make_async_copy(
                    rows[b], acc_sh.at[ibuf[0].at[1, b]], ssem[b]).wait()

        if n_g0 == n_g1:
            run_groups(n_g0)
        else:
            @pl.when(c == 0)
            def _():
                run_groups(n_g0)

            @pl.when(c == 1)
            def _():
                run_groups(n_g1)

        plsc.subcore_barrier()
        pltpu.sync_copy(acc_sh.at[pl.ds(s * RPT, RPT)],
                        out_hbm.at[c, pl.ds(s * RPT, RPT)])

    return agg_kernel


_BLK = 1024


def _mm1_body(x_ref, w_ref, deg_ref, y_ref, dinv_ref):
    dg = deg_ref[...]
    dinv = lax.rsqrt(dg[0] + dg[1] + 1.0)
    xw = jnp.dot(x_ref[...], w_ref[...], preferred_element_type=jnp.float32)
    y_ref[...] = dinv * xw
    dinv_ref[...] = dinv


def _mm2_body(p_ref, y1_ref, dinv_ref, b1_ref, w2_ref, y2_ref):
    pr = p_ref[...]
    dinv = dinv_ref[...]
    agg = pr[0] + pr[1] + y1_ref[...]
    h = jnp.maximum(dinv * agg + b1_ref[...], 0.0)
    y2_ref[...] = dinv * jnp.dot(h, w2_ref[...], preferred_element_type=jnp.float32)


def _fin_body(q_ref, y2_ref, dinv_ref, b2_ref, o_ref):
    qr = q_ref[...]
    o_ref[...] = dinv_ref[...] * (qr[0] + qr[1] + y2_ref[...]) + b2_ref[...]


def _mm1(xp, w1, degp):
    g = N_PAD // _BLK
    din = xp.shape[1]
    return pl.pallas_call(
        _mm1_body,
        grid=(g,),
        in_specs=[
            pl.BlockSpec((_BLK, din), lambda i: (i, 0)),
            pl.BlockSpec((din, w1.shape[1]), lambda i: (0, 0)),
            pl.BlockSpec((NC, _BLK, 1), lambda i: (0, i, 0)),
        ],
        out_specs=[
            pl.BlockSpec((_BLK, w1.shape[1]), lambda i: (i, 0)),
            pl.BlockSpec((_BLK, 1), lambda i: (i, 0)),
        ],
        out_shape=[
            jax.ShapeDtypeStruct((N_PAD, w1.shape[1]), jnp.float32),
            jax.ShapeDtypeStruct((N_PAD, 1), jnp.float32),
        ],
    )(xp, w1, degp)


def _mm2(p, y1, dinv, b1, w2p):
    g = N_PAD // _BLK
    h = y1.shape[1]
    d2 = w2p.shape[1]
    return pl.pallas_call(
        _mm2_body,
        grid=(g,),
        in_specs=[
            pl.BlockSpec((NC, _BLK, h), lambda i: (0, i, 0)),
            pl.BlockSpec((_BLK, h), lambda i: (i, 0)),
            pl.BlockSpec((_BLK, 1), lambda i: (i, 0)),
            pl.BlockSpec((1, h), lambda i: (0, 0)),
            pl.BlockSpec((h, d2), lambda i: (0, 0)),
        ],
        out_specs=pl.BlockSpec((_BLK, d2), lambda i: (i, 0)),
        out_shape=jax.ShapeDtypeStruct((N_PAD, d2), jnp.float32),
    )(p, y1, dinv, b1, w2p)


def _fin(q, y2, dinv, b2p):
    g = N_PAD // _BLK
    d2 = y2.shape[1]
    return pl.pallas_call(
        _fin_body,
        grid=(g,),
        in_specs=[
            pl.BlockSpec((NC, _BLK, d2), lambda i: (0, i, 0)),
            pl.BlockSpec((_BLK, d2), lambda i: (i, 0)),
            pl.BlockSpec((_BLK, 1), lambda i: (i, 0)),
            pl.BlockSpec((1, d2), lambda i: (0, 0)),
        ],
        out_specs=pl.BlockSpec((_BLK, d2), lambda i: (i, 0)),
        out_shape=jax.ShapeDtypeStruct((N_PAD, d2), jnp.float32),
    )(q, y2, dinv, b2p)


@jax.jit
def kernel(x, edge_index, W1, b1, W2, b2):
    n, din = x.shape
    hid = W1.shape[1]
    ncls = W2.shape[1]
    e = edge_index.shape[1]

    src = edge_index[0].astype(jnp.int32)
    dst = edge_index[1].astype(jnp.int32)

    def pack(chunk, nbuf, n_g0, n_g1):
        n_gmax = max(n_g0, n_g1)
        e0 = NS * n_g0 * nbuf * chunk
        e1 = NS * n_g1 * nbuf * chunk
        assert e0 + e1 >= e

        def part(a):
            ap = jnp.concatenate([a, jnp.full((e0 + e1 - e,), n, jnp.int32)])
            a0 = ap[:e0].reshape(NS, n_g0, nbuf, chunk)
            a1 = ap[e0:].reshape(NS, n_g1, nbuf, chunk)
            a0 = jnp.pad(a0, ((0, 0), (0, n_gmax - n_g0), (0, 0), (0, 0)))
            a1 = jnp.pad(a1, ((0, 0), (0, n_gmax - n_g1), (0, 0), (0, 0)))
            return jnp.stack([a0, a1])

        return jnp.stack([part(src), part(dst)], axis=3)

    pk1 = pack(CHUNK1, NBUF1, *SPLIT1)
    pk2 = pack(CHUNK2, NBUF2, *SPLIT2)
    dquant = NW * CHUNK
    e_padd = ((e + dquant - 1) // dquant) * dquant
    dpad = jnp.concatenate([dst, jnp.full((e_padd - e,), n, jnp.int32)])
    dstd = dpad.reshape(NW, e_padd // (NW * CHUNK), CHUNK)

    xp = jnp.zeros((N_PAD, din), jnp.float32).at[:n].set(x)
    d2 = 64
    w2p = jnp.zeros((hid, d2), jnp.float32).at[:, :ncls].set(W2)
    b1r = b1.reshape(1, hid)
    b2p = jnp.zeros((1, d2), jnp.float32).at[0, :ncls].set(b2)

    degp = _make_deg_kernel(e_padd)(dstd)
    y1, dinv = _mm1(xp, W1, degp.reshape(NC, N_PAD, 1))
    p = _make_agg_kernel(hid, CHUNK1, NBUF1, *SPLIT1)(y1, pk1)
    y2 = _mm2(p, y1, dinv, b1r, w2p)
    q = _make_agg_kernel(d2, CHUNK2, NBUF2, *SPLIT2)(y2, pk2)
    o = _fin(q, y2, dinv, b2p)
    return o[:n, :ncls]
